# Initial kernel scaffold; baseline (speedup 1.0000x reference)
import jax
import jax.numpy as jnp
from jax.experimental import pallas as pl


def kernel(box_cls, box_pred, scale_factor, resized_shape):
    raise NotImplementedError("write your pallas kernel here")


# speedup vs baseline: 4.8029x; 4.8029x over previous
import functools

import jax
import jax.numpy as jnp
from jax.experimental import pallas as pl
from jax.experimental.pallas import tpu as pltpu

_K = 300
_ROWS = 12500
_STRIPE = 125
_NCHUNK = 100
_LANES = 128
_C = 80


def _body(cls_ref, bp_ref, sf_ref, rs_ref, out_ref, a_ref, cm_ref):
    a_ref[...] = cls_ref[...]

    def init_cm(i, _):
        blk = a_ref[pl.ds(i * _STRIPE, _STRIPE), :]
        cm_ref[pl.ds(i, 1), :] = jnp.max(blk, axis=0, keepdims=True)
        return 0

    jax.lax.fori_loop(0, _NCHUNK, init_cm, 0)

    b = pl.program_id(0)
    img_h = rs_ref[b, 0]
    img_w = rs_ref[b, 1]
    s0 = img_w / sf_ref[b, 0]
    s1 = img_h / sf_ref[b, 1]
    s2 = img_w / sf_ref[b, 2]
    s3 = img_h / sf_ref[b, 3]

    rio = jax.lax.broadcasted_iota(jnp.int32, (_NCHUNK, _LANES), 0)
    gio = (jax.lax.broadcasted_iota(jnp.int32, (_STRIPE, _LANES), 0) * _LANES
           + jax.lax.broadcasted_iota(jnp.int32, (_STRIPE, _LANES), 1))
    big = jnp.int32(1 << 30)

    def step(it, _):
        cm = cm_ref[...]
        m = jnp.max(cm)
        imin = jnp.min(jnp.where(cm == m, rio, jnp.int32(_NCHUNK)))
        blk = a_ref[pl.ds(imin * _STRIPE, _STRIPE), :]
        gloc = jnp.min(jnp.where(blk == m, gio, big))
        g = imin * (_STRIPE * _LANES) + gloc

        blk2 = jnp.where(gio == gloc, -jnp.inf, blk)
        a_ref[pl.ds(imin * _STRIPE, _STRIPE), :] = blk2
        cm_ref[pl.ds(imin, 1), :] = jnp.max(blk2, axis=0, keepdims=True)

        n = g // _C
        c = g - n * _C

        brow = bp_ref[pl.ds(n, 1), :]
        bs = jax.nn.sigmoid(brow)
        cx = bs[0, 0]
        cy = bs[0, 1]
        w = bs[0, 2]
        h = bs[0, 3]
        x1 = (cx - 0.5 * w) * s0
        y1 = (cy - 0.5 * h) * s1
        x2 = (cx + 0.5 * w) * s2
        y2 = (cy + 0.5 * h) * s3
        score = jax.nn.sigmoid(m)
        label = c.astype(jnp.float32)
        row = jnp.concatenate(
            [x1.reshape(1, 1), y1.reshape(1, 1), x2.reshape(1, 1),
             y2.reshape(1, 1), score.reshape(1, 1), label.reshape(1, 1)],
            axis=1)
        out_ref[pl.ds(it, 1), pl.ds(0, 6)] = row
        return 0

    jax.lax.fori_loop(0, _K, step, 0)


@jax.jit
def kernel(box_cls, box_pred, scale_factor, resized_shape):
    B, N, C = box_cls.shape
    cls2 = box_cls.reshape(B, _ROWS, _LANES)
    rs = resized_shape.astype(jnp.float32)
    return pl.pallas_call(
        _body,
        grid=(B,),
        in_specs=[
            pl.BlockSpec((None, _ROWS, _LANES), lambda b: (b, 0, 0)),
            pl.BlockSpec((None, N, 4), lambda b: (b, 0, 0)),
            pl.BlockSpec((16, 4), lambda b: (0, 0)),
            pl.BlockSpec((16, 2), lambda b: (0, 0)),
        ],
        out_specs=pl.BlockSpec((None, _K, 6), lambda b: (b, 0, 0)),
        out_shape=jax.ShapeDtypeStruct((B, _K, 6), jnp.float32),
        scratch_shapes=[
            pltpu.VMEM((_ROWS, _LANES), jnp.float32),
            pltpu.VMEM((_NCHUNK, _LANES), jnp.float32),
        ],
    )(cls2, box_pred, scale_factor, rs)
